# baseline (speedup 1.0000x reference)
import functools

import jax
import jax.numpy as jnp
from jax import lax
from jax.experimental import pallas as pl
from jax.experimental.pallas import tpu as pltpu
from jax.experimental.pallas import tpu_sc as plsc

E = 8
H = 1024
F = 4096
S = 2048

T = 256
NW = S // T + E - 1
S_PAD = NW * T
FBLK = 1024
NF = F // FBLK

SC_NC = 2
SC_NS = 16
SC_W = SC_NC * SC_NS


def _router_body(x_ref, w_ref, b_ref, assign_ref, w0_ref):
    x = x_ref[...]
    logits = lax.dot_general(x, w_ref[...], (((1,), (1,)), ((), ())),
                             preferred_element_type=jnp.float32)
    logits = logits + b_ref[...]
    m1 = jnp.max(logits, axis=-1, keepdims=True)
    a1 = jnp.argmax(logits, axis=-1, keepdims=True).astype(jnp.int32)
    col = lax.broadcasted_iota(jnp.int32, logits.shape, 1)
    masked = jnp.where(col == a1, -jnp.inf, logits)
    m2 = jnp.max(masked, axis=-1, keepdims=True)
    assign_ref[...] = a1
    w0_ref[...] = 1.0 / (1.0 + jnp.exp(m2 - m1))


def _router(x, router_w, router_b):
    rt = 512
    assign, w0 = pl.pallas_call(
        _router_body,
        grid=(S // rt,),
        in_specs=[
            pl.BlockSpec((rt, H), lambda i: (i, 0)),
            pl.BlockSpec((E, H), lambda i: (0, 0)),
            pl.BlockSpec((1, E), lambda i: (0, 0)),
        ],
        out_specs=[
            pl.BlockSpec((rt, 1), lambda i: (i, 0)),
            pl.BlockSpec((rt, 1), lambda i: (i, 0)),
        ],
        out_shape=[
            jax.ShapeDtypeStruct((S, 1), jnp.int32),
            jax.ShapeDtypeStruct((S, 1), jnp.float32),
        ],
    )(x, router_w, router_b.reshape(1, E))
    return assign[:, 0], w0[:, 0]


def _build_metadata(assign, w0):
    counts = jnp.bincount(assign, length=E).astype(jnp.int32)
    pitems = (counts + T - 1) // T
    cum_items = jnp.cumsum(pitems)
    item_expert = jnp.searchsorted(
        cum_items, jnp.arange(NW, dtype=jnp.int32), side="right"
    ).astype(jnp.int32)
    item_expert = jnp.minimum(item_expert, E - 1)
    pstart_rows = (cum_items - pitems) * T
    cstart = jnp.cumsum(counts) - counts
    order = jnp.argsort(assign, stable=True).astype(jnp.int32)
    sorted_e = assign[order]
    rank = jnp.arange(S, dtype=jnp.int32) - cstart[sorted_e]
    ppos = pstart_rows[sorted_e] + rank
    idx = jnp.zeros(S_PAD, jnp.int32).at[ppos].set(order)
    pos = jnp.zeros(S, jnp.int32).at[order].set(ppos)
    w0_pad = jnp.zeros(S_PAD, jnp.float32).at[ppos].set(w0[order])
    return item_expert, idx, pos, w0_pad


@functools.lru_cache(maxsize=None)
def _make_sc_gather(nrows, ch):
    b_per_w = nrows // SC_W
    nch = b_per_w // ch
    assert b_per_w % ch == 0
    mesh = plsc.VectorSubcoreMesh(core_axis_name="c", subcore_axis_name="s")

    @functools.partial(
        pl.kernel,
        mesh=mesh,
        out_type=jax.ShapeDtypeStruct((nrows, H), jnp.float32),
        scratch_types=[
            pltpu.VMEM((b_per_w,), jnp.int32),
            pltpu.VMEM((ch, H), jnp.float32),
            pltpu.VMEM((ch, H), jnp.float32),
            pltpu.SemaphoreType.DMA,
            pltpu.SemaphoreType.DMA,
        ],
    )
    def k(table_hbm, idx_hbm, out_hbm, idx_v, r0, r1, s0, s1):
        wid = lax.axis_index("s") * SC_NC + lax.axis_index("c")
        base = wid * b_per_w
        bufs, sems = (r0, r1), (s0, s1)
        pltpu.sync_copy(idx_hbm.at[pl.ds(base, b_per_w)], idx_v)
        pltpu.async_copy(table_hbm.at[idx_v.at[pl.ds(0, ch)]], r0, s0)
        for c in range(nch):
            if c + 1 < nch:
                pltpu.async_copy(
                    table_hbm.at[idx_v.at[pl.ds((c + 1) * ch, ch)]],
                    bufs[(c + 1) % 2], sems[(c + 1) % 2])
            pltpu.make_async_copy(
                table_hbm.at[idx_v.at[pl.ds(c * ch, ch)]],
                bufs[c % 2], sems[c % 2]).wait()
            pltpu.sync_copy(bufs[c % 2], out_hbm.at[pl.ds(base + c * ch, ch)])

    return k


def _ffn_body(ie_ref, x_ref, fc1w_ref, fc1b_ref, fc2w_ref, fc2b_ref, w0_ref,
              out_ref):
    del ie_ref
    f = pl.program_id(1)
    x = x_ref[...]
    h = lax.dot_general(x, fc1w_ref[0], (((1,), (1,)), ((), ())),
                        preferred_element_type=jnp.float32)
    h = h + fc1b_ref[0]
    g = 0.5 * h * (1.0 + lax.erf(h * 0.7071067811865476))
    y = lax.dot_general(g, fc2w_ref[0], (((1,), (1,)), ((), ())),
                        preferred_element_type=jnp.float32)

    @pl.when(f == 0)
    def _():
        out_ref[...] = y

    @pl.when(f > 0)
    def _():
        out_ref[...] = out_ref[...] + y

    @pl.when(f == NF - 1)
    def _():
        out_ref[...] = (out_ref[...] + fc2b_ref[0]) * w0_ref[...]


def _ffn(item_expert, x_pad, fc1_w, fc1_b, fc2_w, fc2_b, w0_pad):
    grid_spec = pltpu.PrefetchScalarGridSpec(
        num_scalar_prefetch=1,
        grid=(NW, NF),
        in_specs=[
            pl.BlockSpec((T, H), lambda w, f, ie: (w, 0)),
            pl.BlockSpec((1, FBLK, H), lambda w, f, ie: (ie[w], f, 0)),
            pl.BlockSpec((1, 1, FBLK), lambda w, f, ie: (ie[w], 0, f)),
            pl.BlockSpec((1, H, FBLK), lambda w, f, ie: (ie[w], 0, f)),
            pl.BlockSpec((1, 1, H), lambda w, f, ie: (ie[w], 0, 0)),
            pl.BlockSpec((T, 1), lambda w, f, ie: (w, 0)),
        ],
        out_specs=pl.BlockSpec((T, H), lambda w, f, ie: (w, 0)),
    )
    return pl.pallas_call(
        _ffn_body,
        grid_spec=grid_spec,
        out_shape=jax.ShapeDtypeStruct((S_PAD, H), jnp.float32),
        compiler_params=pltpu.CompilerParams(
            dimension_semantics=("arbitrary", "arbitrary")),
    )(item_expert, x_pad, fc1_w, fc1_b.reshape(E, 1, F), fc2_w,
      fc2_b.reshape(E, 1, H), w0_pad.reshape(S_PAD, 1))


def kernel(hidden_states, router_w, router_b, fc1_w, fc1_b, fc2_w, fc2_b):
    b, s, h = hidden_states.shape
    x = hidden_states.reshape(s, h)
    assign, w0 = _router(x, router_w, router_b)
    item_expert, idx, pos, w0_pad = _build_metadata(assign, w0)
    x_pad = _make_sc_gather(S_PAD, 40)(x, idx)
    y_pad = _ffn(item_expert, x_pad, fc1_w, fc1_b, fc2_w, fc2_b, w0_pad)
    out = _make_sc_gather(S, 32)(y_pad, pos)
    return out.reshape(b, s, h)


# speedup vs baseline: 1.4412x; 1.4412x over previous
import functools

import jax
import jax.numpy as jnp
from jax import lax
from jax.experimental import pallas as pl
from jax.experimental.pallas import tpu as pltpu
from jax.experimental.pallas import tpu_sc as plsc

E = 8
H = 1024
F = 4096
S = 2048

T = 256
NW = S // T + E - 1
S_PAD = NW * T
FBLK = 1024
NF = F // FBLK

SC_NC = 2
SC_NS = 16
SC_W = SC_NC * SC_NS


def _router_body(x_ref, w_ref, b_ref, assign_ref, w0_ref):
    x = x_ref[...]
    logits = lax.dot_general(x, w_ref[...], (((1,), (1,)), ((), ())),
                             preferred_element_type=jnp.float32)
    logits = logits + b_ref[...]
    m1 = jnp.max(logits, axis=-1, keepdims=True)
    a1 = jnp.argmax(logits, axis=-1, keepdims=True).astype(jnp.int32)
    col = lax.broadcasted_iota(jnp.int32, logits.shape, 1)
    masked = jnp.where(col == a1, -jnp.inf, logits)
    m2 = jnp.max(masked, axis=-1, keepdims=True)
    assign_ref[...] = a1
    w0_ref[...] = 1.0 / (1.0 + jnp.exp(m2 - m1))


def _router(x, router_w, router_b):
    rt = 512
    assign, w0 = pl.pallas_call(
        _router_body,
        grid=(S // rt,),
        in_specs=[
            pl.BlockSpec((rt, H), lambda i: (i, 0)),
            pl.BlockSpec((E, H), lambda i: (0, 0)),
            pl.BlockSpec((1, E), lambda i: (0, 0)),
        ],
        out_specs=[
            pl.BlockSpec((rt, 1), lambda i: (i, 0)),
            pl.BlockSpec((rt, 1), lambda i: (i, 0)),
        ],
        out_shape=[
            jax.ShapeDtypeStruct((S, 1), jnp.int32),
            jax.ShapeDtypeStruct((S, 1), jnp.float32),
        ],
    )(x, router_w, router_b.reshape(1, E))
    return assign[:, 0], w0[:, 0]


def _build_metadata(assign, w0):
    oh = (assign[:, None] == jnp.arange(E, dtype=jnp.int32)[None, :])
    csum = jnp.cumsum(oh.astype(jnp.int32), axis=0)
    counts = csum[-1]
    rank = jnp.take_along_axis(csum, assign[:, None], axis=1)[:, 0] - 1
    pitems = (counts + T - 1) // T
    cum_items = jnp.cumsum(pitems)
    item_expert = jnp.searchsorted(
        cum_items, jnp.arange(NW, dtype=jnp.int32), side="right"
    ).astype(jnp.int32)
    item_expert = jnp.minimum(item_expert, E - 1)
    pstart_rows = (cum_items - pitems) * T
    pos = pstart_rows[assign] + rank
    w0_pad = jnp.zeros(S_PAD, jnp.float32).at[pos].set(w0)
    return item_expert, pos, w0_pad


@functools.lru_cache(maxsize=None)
def _make_sc_scatter(nrows_in, nrows_out, ch):
    b_per_w = nrows_in // SC_W
    nch = b_per_w // ch
    assert b_per_w % ch == 0 and nch == 2
    mesh = plsc.VectorSubcoreMesh(core_axis_name="c", subcore_axis_name="s")

    @functools.partial(
        pl.kernel,
        mesh=mesh,
        out_type=jax.ShapeDtypeStruct((nrows_out, H), jnp.float32),
        scratch_types=[
            pltpu.VMEM((ch,), jnp.int32),
            pltpu.VMEM((ch,), jnp.int32),
            pltpu.VMEM((ch, H), jnp.float32),
            pltpu.VMEM((ch, H), jnp.float32),
            pltpu.SemaphoreType.DMA,
            pltpu.SemaphoreType.DMA,
        ],
    )
    def k(src_hbm, pos_hbm, out_hbm, i0, i1, r0, r1, s0, s1):
        wid = lax.axis_index("s") * SC_NC + lax.axis_index("c")
        base = wid * b_per_w
        ivs, rvs, sems = (i0, i1), (r0, r1), (s0, s1)
        for c in range(nch):
            pltpu.sync_copy(pos_hbm.at[pl.ds(base + c * ch, ch)], ivs[c])
            pltpu.sync_copy(src_hbm.at[pl.ds(base + c * ch, ch)], rvs[c])
            pltpu.async_copy(rvs[c], out_hbm.at[ivs[c]], sems[c])
        for c in range(nch):
            pltpu.make_async_copy(rvs[c], out_hbm.at[ivs[c]], sems[c]).wait()

    return k


@functools.lru_cache(maxsize=None)
def _make_sc_gather(nrows, ch):
    b_per_w = nrows // SC_W
    nch = b_per_w // ch
    assert b_per_w % ch == 0
    mesh = plsc.VectorSubcoreMesh(core_axis_name="c", subcore_axis_name="s")

    @functools.partial(
        pl.kernel,
        mesh=mesh,
        out_type=jax.ShapeDtypeStruct((nrows, H), jnp.float32),
        scratch_types=[
            pltpu.VMEM((b_per_w,), jnp.int32),
            pltpu.VMEM((ch, H), jnp.float32),
            pltpu.VMEM((ch, H), jnp.float32),
            pltpu.SemaphoreType.DMA,
            pltpu.SemaphoreType.DMA,
        ],
    )
    def k(table_hbm, idx_hbm, out_hbm, idx_v, r0, r1, s0, s1):
        wid = lax.axis_index("s") * SC_NC + lax.axis_index("c")
        base = wid * b_per_w
        bufs, sems = (r0, r1), (s0, s1)
        pltpu.sync_copy(idx_hbm.at[pl.ds(base, b_per_w)], idx_v)
        pltpu.async_copy(table_hbm.at[idx_v.at[pl.ds(0, ch)]], r0, s0)
        for c in range(nch):
            if c + 1 < nch:
                pltpu.async_copy(
                    table_hbm.at[idx_v.at[pl.ds((c + 1) * ch, ch)]],
                    bufs[(c + 1) % 2], sems[(c + 1) % 2])
            pltpu.make_async_copy(
                table_hbm.at[idx_v.at[pl.ds(c * ch, ch)]],
                bufs[c % 2], sems[c % 2]).wait()
            pltpu.sync_copy(bufs[c % 2], out_hbm.at[pl.ds(base + c * ch, ch)])

    return k


def _ffn_body(ie_ref, x_ref, fc1w_ref, fc1b_ref, fc2w_ref, fc2b_ref, w0_ref,
              out_ref):
    del ie_ref
    f = pl.program_id(1)
    x = x_ref[...]
    h = lax.dot_general(x, fc1w_ref[0], (((1,), (1,)), ((), ())),
                        preferred_element_type=jnp.float32)
    h = h + fc1b_ref[0]
    g = 0.5 * h * (1.0 + lax.erf(h * 0.7071067811865476))
    y = lax.dot_general(g, fc2w_ref[0], (((1,), (1,)), ((), ())),
                        preferred_element_type=jnp.float32)

    @pl.when(f == 0)
    def _():
        out_ref[...] = y

    @pl.when(f > 0)
    def _():
        out_ref[...] = out_ref[...] + y

    @pl.when(f == NF - 1)
    def _():
        out_ref[...] = (out_ref[...] + fc2b_ref[0]) * w0_ref[...]


def _ffn(item_expert, x_pad, fc1_w, fc1_b, fc2_w, fc2_b, w0_pad):
    grid_spec = pltpu.PrefetchScalarGridSpec(
        num_scalar_prefetch=1,
        grid=(NW, NF),
        in_specs=[
            pl.BlockSpec((T, H), lambda w, f, ie: (w, 0)),
            pl.BlockSpec((1, FBLK, H), lambda w, f, ie: (ie[w], f, 0)),
            pl.BlockSpec((1, 1, FBLK), lambda w, f, ie: (ie[w], 0, f)),
            pl.BlockSpec((1, H, FBLK), lambda w, f, ie: (ie[w], 0, f)),
            pl.BlockSpec((1, 1, H), lambda w, f, ie: (ie[w], 0, 0)),
            pl.BlockSpec((T, 1), lambda w, f, ie: (w, 0)),
        ],
        out_specs=pl.BlockSpec((T, H), lambda w, f, ie: (w, 0)),
    )
    return pl.pallas_call(
        _ffn_body,
        grid_spec=grid_spec,
        out_shape=jax.ShapeDtypeStruct((S_PAD, H), jnp.float32),
        compiler_params=pltpu.CompilerParams(
            dimension_semantics=("arbitrary", "arbitrary")),
    )(item_expert, x_pad, fc1_w, fc1_b.reshape(E, 1, F), fc2_w,
      fc2_b.reshape(E, 1, H), w0_pad.reshape(S_PAD, 1))


def kernel(hidden_states, router_w, router_b, fc1_w, fc1_b, fc2_w, fc2_b):
    b, s, h = hidden_states.shape
    x = hidden_states.reshape(s, h)
    assign, w0 = _router(x, router_w, router_b)
    item_expert, pos, w0_pad = _build_metadata(assign, w0)
    x_pad = _make_sc_scatter(S, S_PAD, 32)(x, pos)
    y_pad = _ffn(item_expert, x_pad, fc1_w, fc1_b, fc2_w, fc2_b, w0_pad)
    out = _make_sc_gather(S, 32)(y_pad, pos)
    return out.reshape(b, s, h)


# speedup vs baseline: 1.4962x; 1.0381x over previous
import functools

import jax
import jax.numpy as jnp
from jax import lax
from jax.experimental import pallas as pl
from jax.experimental.pallas import tpu as pltpu
from jax.experimental.pallas import tpu_sc as plsc

E = 8
H = 1024
F = 4096
S = 2048

T = 256
NW = S // T + E - 1
S_PAD = NW * T
FBLK = 1024
NF = F // FBLK

SC_NC = 2
SC_NS = 16
SC_W = SC_NC * SC_NS


def _router_body(x_ref, w_ref, b_ref, assign_ref, w0_ref):
    x = x_ref[...]
    logits = lax.dot_general(x, w_ref[...], (((1,), (1,)), ((), ())),
                             preferred_element_type=jnp.float32)
    logits = logits + b_ref[...]
    m1 = jnp.max(logits, axis=-1, keepdims=True)
    a1 = jnp.argmax(logits, axis=-1, keepdims=True).astype(jnp.int32)
    col = lax.broadcasted_iota(jnp.int32, logits.shape, 1)
    masked = jnp.where(col == a1, -jnp.inf, logits)
    m2 = jnp.max(masked, axis=-1, keepdims=True)
    assign_ref[...] = a1
    w0_ref[...] = 1.0 / (1.0 + jnp.exp(m2 - m1))


def _router(x, router_w, router_b):
    rt = 512
    assign, w0 = pl.pallas_call(
        _router_body,
        grid=(S // rt,),
        in_specs=[
            pl.BlockSpec((rt, H), lambda i: (i, 0)),
            pl.BlockSpec((E, H), lambda i: (0, 0)),
            pl.BlockSpec((1, E), lambda i: (0, 0)),
        ],
        out_specs=[
            pl.BlockSpec((rt, 1), lambda i: (i, 0)),
            pl.BlockSpec((rt, 1), lambda i: (i, 0)),
        ],
        out_shape=[
            jax.ShapeDtypeStruct((S, 1), jnp.int32),
            jax.ShapeDtypeStruct((S, 1), jnp.float32),
        ],
    )(x, router_w, router_b.reshape(1, E))
    return assign[:, 0], w0[:, 0]


def _build_metadata(assign, w0):
    oh = (assign[:, None] == jnp.arange(E, dtype=jnp.int32)[None, :])
    csum = jnp.cumsum(oh.astype(jnp.int32), axis=0)
    counts = csum[-1]
    rank = jnp.take_along_axis(csum, assign[:, None], axis=1)[:, 0] - 1
    pitems = (counts + T - 1) // T
    cum_items = jnp.cumsum(pitems)
    item_expert = jnp.searchsorted(
        cum_items, jnp.arange(NW, dtype=jnp.int32), side="right"
    ).astype(jnp.int32)
    item_expert = jnp.minimum(item_expert, E - 1)
    pstart_rows = (cum_items - pitems) * T
    pos = pstart_rows[assign] + rank
    w0_pad = jnp.zeros(S_PAD, jnp.float32).at[pos].set(w0)
    return item_expert, pos, w0_pad


@functools.lru_cache(maxsize=None)
def _make_sc_scatter(nrows_in, nrows_out, ch):
    b_per_w = nrows_in // SC_W
    nch = b_per_w // ch
    assert b_per_w % ch == 0 and nch == 2
    mesh = plsc.VectorSubcoreMesh(core_axis_name="c", subcore_axis_name="s")

    @functools.partial(
        pl.kernel,
        mesh=mesh,
        out_type=jax.ShapeDtypeStruct((nrows_out, H), jnp.float32),
        scratch_types=[
            pltpu.VMEM((ch,), jnp.int32),
            pltpu.VMEM((ch,), jnp.int32),
            pltpu.VMEM((ch, H), jnp.float32),
            pltpu.VMEM((ch, H), jnp.float32),
            pltpu.SemaphoreType.DMA,
            pltpu.SemaphoreType.DMA,
        ],
    )
    def k(src_hbm, pos_hbm, out_hbm, i0, i1, r0, r1, s0, s1):
        wid = lax.axis_index("s") * SC_NC + lax.axis_index("c")
        base = wid * b_per_w
        ivs, rvs, sems = (i0, i1), (r0, r1), (s0, s1)
        for c in range(nch):
            pltpu.sync_copy(pos_hbm.at[pl.ds(base + c * ch, ch)], ivs[c])
            pltpu.sync_copy(src_hbm.at[pl.ds(base + c * ch, ch)], rvs[c])
            pltpu.async_copy(rvs[c], out_hbm.at[ivs[c]], sems[c])
        for c in range(nch):
            pltpu.make_async_copy(rvs[c], out_hbm.at[ivs[c]], sems[c]).wait()

    return k


@functools.lru_cache(maxsize=None)
def _make_sc_gather(nrows, ch):
    b_per_w = nrows // SC_W
    nch = b_per_w // ch
    assert b_per_w % ch == 0
    mesh = plsc.VectorSubcoreMesh(core_axis_name="c", subcore_axis_name="s")

    @functools.partial(
        pl.kernel,
        mesh=mesh,
        out_type=jax.ShapeDtypeStruct((nrows, H), jnp.float32),
        scratch_types=[
            pltpu.VMEM((b_per_w,), jnp.int32),
            pltpu.VMEM((ch, H), jnp.float32),
            pltpu.VMEM((ch, H), jnp.float32),
            pltpu.SemaphoreType.DMA,
            pltpu.SemaphoreType.DMA,
        ],
    )
    def k(table_hbm, idx_hbm, out_hbm, idx_v, r0, r1, s0, s1):
        wid = lax.axis_index("s") * SC_NC + lax.axis_index("c")
        base = wid * b_per_w
        bufs, sems = (r0, r1), (s0, s1)
        pltpu.sync_copy(idx_hbm.at[pl.ds(base, b_per_w)], idx_v)
        pltpu.async_copy(table_hbm.at[idx_v.at[pl.ds(0, ch)]], r0, s0)
        for c in range(nch):
            if c + 1 < nch:
                pltpu.async_copy(
                    table_hbm.at[idx_v.at[pl.ds((c + 1) * ch, ch)]],
                    bufs[(c + 1) % 2], sems[(c + 1) % 2])
            pltpu.make_async_copy(
                table_hbm.at[idx_v.at[pl.ds(c * ch, ch)]],
                bufs[c % 2], sems[c % 2]).wait()
            pltpu.sync_copy(bufs[c % 2], out_hbm.at[pl.ds(base + c * ch, ch)])

    return k


def _ffn_body(ie_ref, x_ref, fc1w_ref, fc1b_ref, fc2w_ref, fc2b_ref, w0_ref,
              out_ref, acc_ref):
    del ie_ref
    f = pl.program_id(0)
    w = pl.program_id(1)
    x = x_ref[...]
    h = lax.dot_general(x, fc1w_ref[0], (((1,), (1,)), ((), ())),
                        preferred_element_type=jnp.float32)
    h = h + fc1b_ref[0]
    g = 0.5 * h * (1.0 + lax.erf(h * 0.7071067811865476))
    y = lax.dot_general(g, fc2w_ref[0], (((1,), (1,)), ((), ())),
                        preferred_element_type=jnp.float32)
    sl = pl.ds(w * T, T)

    @pl.when(f == 0)
    def _():
        acc_ref[sl, :] = y

    @pl.when(jnp.logical_and(f > 0, f < NF - 1))
    def _():
        acc_ref[sl, :] += y

    @pl.when(f == NF - 1)
    def _():
        out_ref[...] = (acc_ref[sl, :] + y + fc2b_ref[0]) * w0_ref[...]


def _ffn(item_expert, x_pad, fc1_w, fc1_b, fc2_w, fc2_b, w0_pad):
    grid_spec = pltpu.PrefetchScalarGridSpec(
        num_scalar_prefetch=1,
        grid=(NF, NW),
        in_specs=[
            pl.BlockSpec((T, H), lambda f, w, ie: (w, 0)),
            pl.BlockSpec((1, FBLK, H), lambda f, w, ie: (ie[w], f, 0)),
            pl.BlockSpec((1, 1, FBLK), lambda f, w, ie: (ie[w], 0, f)),
            pl.BlockSpec((1, H, FBLK), lambda f, w, ie: (ie[w], 0, f)),
            pl.BlockSpec((1, 1, H), lambda f, w, ie: (ie[w], 0, 0)),
            pl.BlockSpec((T, 1), lambda f, w, ie: (w, 0)),
        ],
        out_specs=pl.BlockSpec(
            (T, H), lambda f, w, ie: (jnp.where(f == NF - 1, w, 0), 0)),
        scratch_shapes=[pltpu.VMEM((S_PAD, H), jnp.float32)],
    )
    return pl.pallas_call(
        _ffn_body,
        grid_spec=grid_spec,
        out_shape=jax.ShapeDtypeStruct((S_PAD, H), jnp.float32),
        compiler_params=pltpu.CompilerParams(
            dimension_semantics=("arbitrary", "arbitrary")),
    )(item_expert, x_pad, fc1_w, fc1_b.reshape(E, 1, F), fc2_w,
      fc2_b.reshape(E, 1, H), w0_pad.reshape(S_PAD, 1))


def kernel(hidden_states, router_w, router_b, fc1_w, fc1_b, fc2_w, fc2_b):
    b, s, h = hidden_states.shape
    x = hidden_states.reshape(s, h)
    assign, w0 = _router(x, router_w, router_b)
    item_expert, pos, w0_pad = _build_metadata(assign, w0)
    x_pad = _make_sc_scatter(S, S_PAD, 32)(x, pos)
    y_pad = _ffn(item_expert, x_pad, fc1_w, fc1_b, fc2_w, fc2_b, w0_pad)
    out = _make_sc_gather(S, 32)(y_pad, pos)
    return out.reshape(b, s, h)


# speedup vs baseline: 1.8523x; 1.2380x over previous
import functools

import jax
import jax.numpy as jnp
from jax import lax
from jax.experimental import pallas as pl
from jax.experimental.pallas import tpu as pltpu
from jax.experimental.pallas import tpu_sc as plsc

E = 8
H = 1024
F = 4096
S = 2048

T = 256
NW = S // T + E - 1
S_PAD = NW * T
FBLK = 1024
NF = F // FBLK

SC_NC = 2
SC_NS = 16
SC_W = SC_NC * SC_NS


def _router_body(x_ref, w_ref, b_ref, assign_ref, w0_ref):
    x = x_ref[...]
    logits = lax.dot_general(x, w_ref[...], (((1,), (1,)), ((), ())),
                             preferred_element_type=jnp.float32)
    logits = logits + b_ref[...]
    m1 = jnp.max(logits, axis=-1, keepdims=True)
    a1 = jnp.argmax(logits, axis=-1, keepdims=True).astype(jnp.int32)
    col = lax.broadcasted_iota(jnp.int32, logits.shape, 1)
    masked = jnp.where(col == a1, -jnp.inf, logits)
    m2 = jnp.max(masked, axis=-1, keepdims=True)
    assign_ref[...] = a1
    w0_ref[...] = 1.0 / (1.0 + jnp.exp(m2 - m1))


def _router(x, router_w, router_b):
    rt = 512
    assign, w0 = pl.pallas_call(
        _router_body,
        grid=(S // rt,),
        in_specs=[
            pl.BlockSpec((rt, H), lambda i: (i, 0)),
            pl.BlockSpec((E, H), lambda i: (0, 0)),
            pl.BlockSpec((1, E), lambda i: (0, 0)),
        ],
        out_specs=[
            pl.BlockSpec((rt, 1), lambda i: (i, 0)),
            pl.BlockSpec((rt, 1), lambda i: (i, 0)),
        ],
        out_shape=[
            jax.ShapeDtypeStruct((S, 1), jnp.int32),
            jax.ShapeDtypeStruct((S, 1), jnp.float32),
        ],
    )(x, router_w, router_b.reshape(1, E))
    return assign[:, 0], w0[:, 0]


def _build_metadata(assign, w0):
    oh = (assign[:, None] == jnp.arange(E, dtype=jnp.int32)[None, :])
    csum = jnp.cumsum(oh.astype(jnp.int32), axis=0)
    counts = csum[-1]
    rank = jnp.take_along_axis(csum, assign[:, None], axis=1)[:, 0] - 1
    pitems = (counts + T - 1) // T
    cum_items = jnp.cumsum(pitems)
    bstart = jnp.concatenate(
        [jnp.zeros((1,), jnp.int32), cum_items]).astype(jnp.int32)
    pstart_rows = (cum_items - pitems) * T
    pos = pstart_rows[assign] + rank
    w0_pad = jnp.zeros(S_PAD, jnp.float32).at[pos].set(w0)
    return bstart, pos, w0_pad


@functools.lru_cache(maxsize=None)
def _make_sc_scatter(nrows_in, nrows_out, ch):
    b_per_w = nrows_in // SC_W
    nch = b_per_w // ch
    assert b_per_w % ch == 0 and nch == 2
    mesh = plsc.VectorSubcoreMesh(core_axis_name="c", subcore_axis_name="s")

    @functools.partial(
        pl.kernel,
        mesh=mesh,
        out_type=jax.ShapeDtypeStruct((nrows_out, H), jnp.float32),
        scratch_types=[
            pltpu.VMEM((ch,), jnp.int32),
            pltpu.VMEM((ch,), jnp.int32),
            pltpu.VMEM((ch, H), jnp.float32),
            pltpu.VMEM((ch, H), jnp.float32),
            pltpu.SemaphoreType.DMA,
            pltpu.SemaphoreType.DMA,
        ],
    )
    def k(src_hbm, pos_hbm, out_hbm, i0, i1, r0, r1, s0, s1):
        wid = lax.axis_index("s") * SC_NC + lax.axis_index("c")
        base = wid * b_per_w
        ivs, rvs, sems = (i0, i1), (r0, r1), (s0, s1)
        for c in range(nch):
            pltpu.sync_copy(pos_hbm.at[pl.ds(base + c * ch, ch)], ivs[c])
            pltpu.sync_copy(src_hbm.at[pl.ds(base + c * ch, ch)], rvs[c])
            pltpu.async_copy(rvs[c], out_hbm.at[ivs[c]], sems[c])
        for c in range(nch):
            pltpu.make_async_copy(rvs[c], out_hbm.at[ivs[c]], sems[c]).wait()

    return k


@functools.lru_cache(maxsize=None)
def _make_sc_gather(nrows, ch):
    b_per_w = nrows // SC_W
    nch = b_per_w // ch
    assert b_per_w % ch == 0
    mesh = plsc.VectorSubcoreMesh(core_axis_name="c", subcore_axis_name="s")

    @functools.partial(
        pl.kernel,
        mesh=mesh,
        out_type=jax.ShapeDtypeStruct((nrows, H), jnp.float32),
        scratch_types=[
            pltpu.VMEM((b_per_w,), jnp.int32),
            pltpu.VMEM((ch, H), jnp.float32),
            pltpu.VMEM((ch, H), jnp.float32),
            pltpu.SemaphoreType.DMA,
            pltpu.SemaphoreType.DMA,
        ],
    )
    def k(table_hbm, idx_hbm, out_hbm, idx_v, r0, r1, s0, s1):
        wid = lax.axis_index("s") * SC_NC + lax.axis_index("c")
        base = wid * b_per_w
        bufs, sems = (r0, r1), (s0, s1)
        pltpu.sync_copy(idx_hbm.at[pl.ds(base, b_per_w)], idx_v)
        pltpu.async_copy(table_hbm.at[idx_v.at[pl.ds(0, ch)]], r0, s0)
        for c in range(nch):
            if c + 1 < nch:
                pltpu.async_copy(
                    table_hbm.at[idx_v.at[pl.ds((c + 1) * ch, ch)]],
                    bufs[(c + 1) % 2], sems[(c + 1) % 2])
            pltpu.make_async_copy(
                table_hbm.at[idx_v.at[pl.ds(c * ch, ch)]],
                bufs[c % 2], sems[c % 2]).wait()
            pltpu.sync_copy(bufs[c % 2], out_hbm.at[pl.ds(base + c * ch, ch)])

    return k


def _ffn_body(bs_ref, x_ref, fc1w_ref, fc1b_ref, fc2w_ref, fc2b_ref, w0_ref,
              out_ref):
    f = pl.program_id(0)
    e = pl.program_id(1)
    b_lo = bs_ref[e]
    b_hi = bs_ref[e + 1]

    def item(b, carry):
        rows = pl.ds(b * T, T)
        xb = x_ref[rows, :]
        h = lax.dot_general(xb, fc1w_ref[0], (((1,), (1,)), ((), ())),
                            preferred_element_type=jnp.float32)
        h = h + fc1b_ref[0]
        g = 0.5 * h * (1.0 + lax.erf(h * 0.7071067811865476))
        y = lax.dot_general(g, fc2w_ref[0], (((1,), (1,)), ((), ())),
                            preferred_element_type=jnp.float32)

        @pl.when(f == 0)
        def _():
            out_ref[rows, :] = y

        @pl.when(jnp.logical_and(f > 0, f < NF - 1))
        def _():
            out_ref[rows, :] += y

        @pl.when(f == NF - 1)
        def _():
            out_ref[rows, :] = (out_ref[rows, :] + y + fc2b_ref[0]) * w0_ref[rows, :]

        return carry

    lax.fori_loop(b_lo, b_hi, item, 0)


def _ffn(bstart, x_pad, fc1_w, fc1_b, fc2_w, fc2_b, w0_pad):
    grid_spec = pltpu.PrefetchScalarGridSpec(
        num_scalar_prefetch=1,
        grid=(NF, E),
        in_specs=[
            pl.BlockSpec((S_PAD, H), lambda f, e, bs: (0, 0)),
            pl.BlockSpec((1, FBLK, H), lambda f, e, bs: (e, f, 0)),
            pl.BlockSpec((1, 1, FBLK), lambda f, e, bs: (e, 0, f)),
            pl.BlockSpec((1, H, FBLK), lambda f, e, bs: (e, 0, f)),
            pl.BlockSpec((1, 1, H), lambda f, e, bs: (e, 0, 0)),
            pl.BlockSpec((S_PAD, 1), lambda f, e, bs: (0, 0)),
        ],
        out_specs=pl.BlockSpec((S_PAD, H), lambda f, e, bs: (0, 0)),
    )
    return pl.pallas_call(
        _ffn_body,
        grid_spec=grid_spec,
        out_shape=jax.ShapeDtypeStruct((S_PAD, H), jnp.float32),
        compiler_params=pltpu.CompilerParams(
            dimension_semantics=("arbitrary", "arbitrary"),
            vmem_limit_bytes=120 * 1024 * 1024),
    )(bstart, x_pad, fc1_w, fc1_b.reshape(E, 1, F), fc2_w,
      fc2_b.reshape(E, 1, H), w0_pad.reshape(S_PAD, 1))


def kernel(hidden_states, router_w, router_b, fc1_w, fc1_b, fc2_w, fc2_b):
    b, s, h = hidden_states.shape
    x = hidden_states.reshape(s, h)
    assign, w0 = _router(x, router_w, router_b)
    bstart, pos, w0_pad = _build_metadata(assign, w0)
    x_pad = _make_sc_scatter(S, S_PAD, 32)(x, pos)
    y_pad = _ffn(bstart, x_pad, fc1_w, fc1_b, fc2_w, fc2_b, w0_pad)
    out = _make_sc_gather(S, 32)(y_pad, pos)
    return out.reshape(b, s, h)
